# baseline (speedup 1.0000x reference)
import functools

import jax
import jax.numpy as jnp
from jax import lax
from jax.experimental import pallas as pl
from jax.experimental.pallas import tpu as pltpu
from jax.experimental.pallas import tpu_sc as plsc

VOCAB_PAD = 1024
ROW_PITCH = 72
NC = 2
NS = 16
NW = NC * NS
LANES = 16


def _mesh():
    return plsc.VectorSubcoreMesh(
        core_axis_name="c", subcore_axis_name="s",
        num_cores=NC, num_subcores=NS)


def _worker_id():
    return lax.axis_index("s") * NC + lax.axis_index("c")


@functools.lru_cache(maxsize=None)
def _hist_kernel(bw: int):
    @functools.partial(
        pl.kernel,
        out_type=jax.ShapeDtypeStruct((NW, VOCAB_PAD), jnp.float32),
        mesh=_mesh(),
        scratch_types=[
            pltpu.VMEM((bw,), jnp.int32),
            pltpu.VMEM((VOCAB_PAD,), jnp.float32),
        ],
        compiler_params=pltpu.CompilerParams(needs_layout_passes=False),
    )
    def hist(idx_hbm, out_hbm, idx_v, cnt_v):
        wid = _worker_id()
        pltpu.sync_copy(idx_hbm.at[pl.ds(wid * bw, bw)], idx_v)

        def zero_body(i, carry):
            cnt_v[pl.ds(i * LANES, LANES)] = jnp.zeros((LANES,), jnp.float32)
            return carry
        lax.fori_loop(0, VOCAB_PAD // LANES, zero_body, 0)

        ones = jnp.ones((LANES,), jnp.float32)

        def body(i, carry):
            iv = idx_v[pl.ds(i * LANES, LANES)]
            plsc.addupdate_scatter(cnt_v, [iv], ones)
            return carry
        lax.fori_loop(0, bw // LANES, body, 0)

        pltpu.sync_copy(cnt_v, out_hbm.at[wid])

    return hist


@functools.lru_cache(maxsize=None)
def _norm_kernel(d: int, n_elems: float):
    def body(cnt_ref, tab_t_ref, tab_p_ref, out_ref):
        cnt = jnp.sum(cnt_ref[...], axis=0, keepdims=True)
        tab_t = tab_t_ref[...]
        row_sum = jnp.sum(tab_t, axis=0, keepdims=True)
        row_sumsq = jnp.sum(tab_t * tab_t, axis=0, keepdims=True)
        s = jnp.sum(cnt * row_sum)
        q = jnp.sum(cnt * row_sumsq)
        mean = s / n_elems
        var = (q - s * mean) / (n_elems - 1.0)
        scale = lax.rsqrt(var) * (1.0 / 6.0)
        out_ref[...] = jnp.clip(
            (tab_p_ref[...] - mean) * scale + 0.5, 0.0, 1.0)

    return pl.pallas_call(
        body,
        out_shape=jax.ShapeDtypeStruct((VOCAB_PAD, ROW_PITCH), jnp.float32),
    )


@functools.lru_cache(maxsize=None)
def _gather_t_kernel(nb: int, nh: int, d: int):
    ngr = nh * nb // 128
    gpw = ngr // NW
    nbb = nb // 128
    dt = d // 8

    @functools.partial(
        pl.kernel,
        out_type=jax.ShapeDtypeStruct((nh, dt, nbb, 8, 128), jnp.float32),
        mesh=_mesh(),
        scratch_types=[
            pltpu.VMEM((2, 1, 128), jnp.int32),
            pltpu.VMEM((2, 128, ROW_PITCH), jnp.float32),
            pltpu.VMEM((128 * 73,), jnp.float32),
            pltpu.VMEM((2, dt, 8, 128), jnp.float32),
            pltpu.SemaphoreType.DMA,
            pltpu.SemaphoreType.DMA,
            pltpu.SemaphoreType.DMA,
        ],
        compiler_params=pltpu.CompilerParams(
            needs_layout_passes=False, use_tc_tiling_on_sc=False),
    )
    def gather(ntab_hbm, idx_hbm, out_hbm, idx_v, rows_v, rows_o, rowst_v,
               isem, gsem, osem):
        wid = _worker_id()
        g0 = wid * gpw
        iota = lax.iota(jnp.int32, 16)
        rowv73 = [(iota + k * 16) * 73 for k in range(8)]

        def idx_copy(i, b):
            return pltpu.async_copy(
                idx_hbm.at[pl.ds(g0 + i, 1)], idx_v.at[b], isem)

        def wait_idx(i, b):
            pltpu.make_async_copy(
                idx_hbm.at[pl.ds(g0 + i, 1)], idx_v.at[b], isem).wait()

        def fire_gather(b):
            pltpu.async_copy(
                ntab_hbm.at[idx_v.at[b].at[0]], rows_v.at[b], gsem)

        def wait_gather(b):
            pltpu.make_async_copy(
                ntab_hbm.at[idx_v.at[b].at[0]], rows_v.at[b], gsem).wait()

        def out_store(i, b):
            g = g0 + i
            h = g // nbb
            bb = lax.rem(g, nbb)
            return pltpu.async_copy(
                rowst_v.at[b], out_hbm.at[h, :, bb], osem)

        def wait_out_store(i, b):
            g = g0 + i
            h = g // nbb
            bb = lax.rem(g, nbb)
            pltpu.make_async_copy(
                rowst_v.at[b], out_hbm.at[h, :, bb], osem).wait()

        def transpose(b):
            src = rows_v.at[b]

            @plsc.parallel_loop(0, 128, step=1, unroll=8)
            def _(r):
                for k in range(4):
                    rows_o[pl.ds(r * 73 + k * 16, 16)] = (
                        src[r, pl.ds(k * 16, 16)])

            @plsc.parallel_loop(0, d, step=1, unroll=4)
            def _(dd):
                dt_i = dd // 8
                di_i = lax.rem(dd, 8)
                for k in range(8):
                    vals = plsc.load_gather(rows_o, [rowv73[k] + dd])
                    rowst_v[b, dt_i, di_i, pl.ds(k * 16, 16)] = vals

        idx_copy(0, 0).wait()
        fire_gather(0)
        idx_copy(1, 1)

        def pipe(i2, carry):
            for b in range(2):
                other = 1 - b
                i = i2 * 2 + b
                wait_gather(b)

                @pl.when(i + 1 < gpw)
                def _():
                    wait_idx(i + 1, other)
                    fire_gather(other)

                    @pl.when(i + 2 < gpw)
                    def _():
                        idx_copy(i + 2, b)

                @pl.when(i >= 2)
                def _():
                    wait_out_store(i - 2, b)
                transpose(b)
                out_store(i, b)
            return carry

        lax.fori_loop(0, gpw // 2, pipe, 0)
        wait_out_store(gpw - 2, 0)
        wait_out_store(gpw - 1, 1)

    return gather


def kernel(x, table):
    nb, nh = x.shape
    v, d = table.shape
    bt = nb * nh
    bw = bt // NW
    xt = jnp.transpose(x).astype(jnp.int32).reshape(nh * nb // 128, 128)
    xt_flat = xt.reshape(bt)
    tab_pad = jnp.pad(table, ((0, VOCAB_PAD - v), (0, 0)))
    tab_p = jnp.pad(tab_pad, ((0, 0), (0, ROW_PITCH - d)))
    counts = _hist_kernel(bw)(xt_flat)
    ntab = _norm_kernel(d, float(bt) * d)(counts, tab_pad.T, tab_p)
    out5 = _gather_t_kernel(nb, nh, d)(ntab, xt)
    return jnp.transpose(out5, (2, 4, 0, 1, 3)).reshape(nb, nh, d)


# speedup vs baseline: 2.4350x; 2.4350x over previous
import functools

import jax
import jax.numpy as jnp
from jax import lax
from jax.experimental import pallas as pl
from jax.experimental.pallas import tpu as pltpu
from jax.experimental.pallas import tpu_sc as plsc

VOCAB_PAD = 1024
ROW_PITCH = 64
NC = 2
NS = 16
NW = NC * NS
LANES = 16


def _mesh():
    return plsc.VectorSubcoreMesh(
        core_axis_name="c", subcore_axis_name="s",
        num_cores=NC, num_subcores=NS)


def _worker_id():
    return lax.axis_index("s") * NC + lax.axis_index("c")


@functools.lru_cache(maxsize=None)
def _hist_kernel(bw: int):
    @functools.partial(
        pl.kernel,
        out_type=jax.ShapeDtypeStruct((NW, VOCAB_PAD), jnp.float32),
        mesh=_mesh(),
        scratch_types=[
            pltpu.VMEM((bw,), jnp.int32),
            pltpu.VMEM((VOCAB_PAD,), jnp.float32),
        ],
        compiler_params=pltpu.CompilerParams(needs_layout_passes=False),
    )
    def hist(idx_hbm, out_hbm, idx_v, cnt_v):
        wid = _worker_id()
        pltpu.sync_copy(idx_hbm.at[pl.ds(wid * bw, bw)], idx_v)

        def zero_body(i, carry):
            cnt_v[pl.ds(i * LANES, LANES)] = jnp.zeros((LANES,), jnp.float32)
            return carry
        lax.fori_loop(0, VOCAB_PAD // LANES, zero_body, 0)

        ones = jnp.ones((LANES,), jnp.float32)

        def body(i, carry):
            iv = idx_v[pl.ds(i * LANES, LANES)]
            plsc.addupdate_scatter(cnt_v, [iv], ones)
            return carry
        lax.fori_loop(0, bw // LANES, body, 0)

        pltpu.sync_copy(cnt_v, out_hbm.at[wid])

    return hist


@functools.lru_cache(maxsize=None)
def _norm_kernel(d: int, n_elems: float):
    def body(cnt_ref, tab_t_ref, tab_p_ref, out_ref):
        cnt = jnp.sum(cnt_ref[...], axis=0, keepdims=True)
        tab_t = tab_t_ref[...]
        row_sum = jnp.sum(tab_t, axis=0, keepdims=True)
        row_sumsq = jnp.sum(tab_t * tab_t, axis=0, keepdims=True)
        s = jnp.sum(cnt * row_sum)
        q = jnp.sum(cnt * row_sumsq)
        mean = s / n_elems
        var = (q - s * mean) / (n_elems - 1.0)
        scale = lax.rsqrt(var) * (1.0 / 6.0)
        out_ref[...] = jnp.clip(
            (tab_p_ref[...] - mean) * scale + 0.5, 0.0, 1.0)

    return pl.pallas_call(
        body,
        out_shape=jax.ShapeDtypeStruct((VOCAB_PAD, ROW_PITCH), jnp.float32),
    )


@functools.lru_cache(maxsize=None)
def _gather_t_kernel(nb: int, nh: int, d: int):
    ngr = nh * nb // 128
    gpw = ngr // NW
    nbb = nb // 128
    dt = d // 8

    @functools.partial(
        pl.kernel,
        out_type=jax.ShapeDtypeStruct((nh, dt, nbb, 8, 128), jnp.float32),
        mesh=_mesh(),
        scratch_types=[
            pltpu.VMEM((2, 1, 128), jnp.int32),
            pltpu.VMEM((2, 128, ROW_PITCH), jnp.float32),
            pltpu.VMEM((128 * 73,), jnp.float32),
            pltpu.VMEM((2, dt, 8, 128), jnp.float32),
            pltpu.VMEM_SHARED((VOCAB_PAD, ROW_PITCH), jnp.float32),
            pltpu.SemaphoreType.DMA,
            pltpu.SemaphoreType.DMA,
            pltpu.SemaphoreType.DMA,
        ],
        compiler_params=pltpu.CompilerParams(
            needs_layout_passes=False, use_tc_tiling_on_sc=False),
    )
    def gather(ntab_hbm, idx_hbm, out_hbm, idx_v, rows_v, rows_o, rowst_v,
               ntab_sh, isem, gsem, osem):
        wid = _worker_id()
        g0 = wid * gpw

        @pl.when(lax.axis_index("s") == 0)
        def _():
            pltpu.sync_copy(ntab_hbm, ntab_sh)
        plsc.subcore_barrier()
        iota = lax.iota(jnp.int32, 16)
        rowv73 = [(iota + k * 16) * 73 for k in range(8)]

        def idx_copy(i, b):
            return pltpu.async_copy(
                idx_hbm.at[pl.ds(g0 + i, 1)], idx_v.at[b], isem)

        def wait_idx(i, b):
            pltpu.make_async_copy(
                idx_hbm.at[pl.ds(g0 + i, 1)], idx_v.at[b], isem).wait()

        def fire_gather(b):
            pltpu.async_copy(
                ntab_sh.at[idx_v.at[b].at[0]], rows_v.at[b], gsem)

        def wait_gather(b):
            pltpu.make_async_copy(
                ntab_sh.at[idx_v.at[b].at[0]], rows_v.at[b], gsem).wait()

        def out_store(i, b):
            g = g0 + i
            h = g // nbb
            bb = lax.rem(g, nbb)
            return pltpu.async_copy(
                rowst_v.at[b], out_hbm.at[h, :, bb], osem)

        def wait_out_store(i, b):
            g = g0 + i
            h = g // nbb
            bb = lax.rem(g, nbb)
            pltpu.make_async_copy(
                rowst_v.at[b], out_hbm.at[h, :, bb], osem).wait()

        def transpose(b):
            src = rows_v.at[b]

            @plsc.parallel_loop(0, 128, step=1, unroll=8)
            def _(r):
                for k in range(4):
                    rows_o[pl.ds(r * 73 + k * 16, 16)] = (
                        src[r, pl.ds(k * 16, 16)])

            @plsc.parallel_loop(0, d, step=1, unroll=4)
            def _(dd):
                dt_i = dd // 8
                di_i = lax.rem(dd, 8)
                for k in range(8):
                    vals = plsc.load_gather(rows_o, [rowv73[k] + dd])
                    rowst_v[b, dt_i, di_i, pl.ds(k * 16, 16)] = vals

        idx_copy(0, 0).wait()
        fire_gather(0)
        idx_copy(1, 1)

        def pipe(i2, carry):
            for b in range(2):
                other = 1 - b
                i = i2 * 2 + b
                wait_gather(b)

                @pl.when(i + 1 < gpw)
                def _():
                    wait_idx(i + 1, other)
                    fire_gather(other)

                    @pl.when(i + 2 < gpw)
                    def _():
                        idx_copy(i + 2, b)

                @pl.when(i >= 2)
                def _():
                    wait_out_store(i - 2, b)
                transpose(b)
                out_store(i, b)
            return carry

        lax.fori_loop(0, gpw // 2, pipe, 0)
        wait_out_store(gpw - 2, 0)
        wait_out_store(gpw - 1, 1)

    return gather


def kernel(x, table):
    nb, nh = x.shape
    v, d = table.shape
    bt = nb * nh
    bw = bt // NW
    xt = jnp.transpose(x).astype(jnp.int32).reshape(nh * nb // 128, 128)
    xt_flat = xt.reshape(bt)
    tab_pad = jnp.pad(table, ((0, VOCAB_PAD - v), (0, 0)))
    tab_p = jnp.pad(tab_pad, ((0, 0), (0, ROW_PITCH - d))) if ROW_PITCH > d else tab_pad
    counts = _hist_kernel(bw)(xt_flat)
    ntab = _norm_kernel(d, float(bt) * d)(counts, tab_pad.T, tab_p)
    out5 = _gather_t_kernel(nb, nh, d)(ntab, xt)
    return jnp.transpose(out5, (2, 4, 0, 1, 3)).reshape(nb, nh, d)


# speedup vs baseline: 2.5555x; 1.0495x over previous
import functools

import jax
import jax.numpy as jnp
from jax import lax
from jax.experimental import pallas as pl
from jax.experimental.pallas import tpu as pltpu
from jax.experimental.pallas import tpu_sc as plsc

VOCAB_PAD = 1024
ROW_PITCH = 64
NC = 2
NS = 16
NW = NC * NS
LANES = 16


def _mesh():
    return plsc.VectorSubcoreMesh(
        core_axis_name="c", subcore_axis_name="s",
        num_cores=NC, num_subcores=NS)


def _worker_id():
    return lax.axis_index("s") * NC + lax.axis_index("c")


@functools.lru_cache(maxsize=None)
def _hist_kernel(bw: int):
    @functools.partial(
        pl.kernel,
        out_type=jax.ShapeDtypeStruct((NW, VOCAB_PAD), jnp.float32),
        mesh=_mesh(),
        scratch_types=[
            pltpu.VMEM((bw,), jnp.int32),
            pltpu.VMEM((VOCAB_PAD,), jnp.float32),
        ],
        compiler_params=pltpu.CompilerParams(needs_layout_passes=False),
    )
    def hist(idx_hbm, out_hbm, idx_v, cnt_v):
        wid = _worker_id()
        pltpu.sync_copy(idx_hbm.at[pl.ds(wid * bw, bw)], idx_v)

        def zero_body(i, carry):
            cnt_v[pl.ds(i * LANES, LANES)] = jnp.zeros((LANES,), jnp.float32)
            return carry
        lax.fori_loop(0, VOCAB_PAD // LANES, zero_body, 0)

        ones = jnp.ones((LANES,), jnp.float32)

        @plsc.parallel_loop(0, bw // LANES, step=1, unroll=8)
        def _(i):
            iv = idx_v[pl.ds(i * LANES, LANES)]
            plsc.addupdate_scatter(cnt_v, [iv], ones)

        pltpu.sync_copy(cnt_v, out_hbm.at[wid])

    return hist


@functools.lru_cache(maxsize=None)
def _norm_kernel(d: int, n_elems: float):
    def body(cnt_ref, tab_t_ref, tab_p_ref, out_ref):
        cnt = jnp.sum(cnt_ref[...], axis=0, keepdims=True)
        tab_t = tab_t_ref[...]
        row_sum = jnp.sum(tab_t, axis=0, keepdims=True)
        row_sumsq = jnp.sum(tab_t * tab_t, axis=0, keepdims=True)
        s = jnp.sum(cnt * row_sum)
        q = jnp.sum(cnt * row_sumsq)
        mean = s / n_elems
        var = (q - s * mean) / (n_elems - 1.0)
        scale = lax.rsqrt(var) * (1.0 / 6.0)
        out_ref[...] = jnp.clip(
            (tab_p_ref[...] - mean) * scale + 0.5, 0.0, 1.0)

    return pl.pallas_call(
        body,
        out_shape=jax.ShapeDtypeStruct((VOCAB_PAD, ROW_PITCH), jnp.float32),
    )


@functools.lru_cache(maxsize=None)
def _gather_t_kernel(nb: int, nh: int, d: int):
    ngr = nh * nb // 128
    gpw = ngr // NW
    nbb = nb // 128
    dt = d // 8

    @functools.partial(
        pl.kernel,
        out_type=jax.ShapeDtypeStruct((nh, dt, nbb, 8, 128), jnp.float32),
        mesh=_mesh(),
        scratch_types=[
            pltpu.VMEM((2, 1, 128), jnp.int32),
            pltpu.VMEM((2, 128, ROW_PITCH), jnp.float32),
            pltpu.VMEM((128 * 73,), jnp.float32),
            pltpu.VMEM((2, dt, 8, 128), jnp.float32),
            pltpu.VMEM_SHARED((VOCAB_PAD, ROW_PITCH), jnp.float32),
            pltpu.SemaphoreType.DMA,
            pltpu.SemaphoreType.DMA,
            pltpu.SemaphoreType.DMA,
        ],
        compiler_params=pltpu.CompilerParams(
            needs_layout_passes=False, use_tc_tiling_on_sc=False),
    )
    def gather(ntab_hbm, idx_hbm, out_hbm, idx_v, rows_v, rows_o, rowst_v,
               ntab_sh, isem, gsem, osem):
        wid = _worker_id()
        g0 = wid * gpw

        @pl.when(lax.axis_index("s") == 0)
        def _():
            pltpu.sync_copy(ntab_hbm, ntab_sh)
        plsc.subcore_barrier()
        iota = lax.iota(jnp.int32, 16)
        rowv73 = [(iota + k * 16) * 73 for k in range(8)]

        def idx_copy(i, b):
            return pltpu.async_copy(
                idx_hbm.at[pl.ds(g0 + i, 1)], idx_v.at[b], isem)

        def wait_idx(i, b):
            pltpu.make_async_copy(
                idx_hbm.at[pl.ds(g0 + i, 1)], idx_v.at[b], isem).wait()

        def fire_gather(b):
            pltpu.async_copy(
                ntab_sh.at[idx_v.at[b].at[0]], rows_v.at[b], gsem)

        def wait_gather(b):
            pltpu.make_async_copy(
                ntab_sh.at[idx_v.at[b].at[0]], rows_v.at[b], gsem).wait()

        def out_store(i, b):
            g = g0 + i
            h = g // nbb
            bb = lax.rem(g, nbb)
            return pltpu.async_copy(
                rowst_v.at[b], out_hbm.at[h, :, bb], osem)

        def wait_out_store(i, b):
            g = g0 + i
            h = g // nbb
            bb = lax.rem(g, nbb)
            pltpu.make_async_copy(
                rowst_v.at[b], out_hbm.at[h, :, bb], osem).wait()

        def transpose(b):
            src = rows_v.at[b]

            @plsc.parallel_loop(0, 128, step=1, unroll=8)
            def _(r):
                for k in range(4):
                    rows_o[pl.ds(r * 73 + k * 16, 16)] = (
                        src[r, pl.ds(k * 16, 16)])

            @plsc.parallel_loop(0, d, step=1, unroll=4)
            def _(dd):
                dt_i = dd // 8
                di_i = lax.rem(dd, 8)
                for k in range(8):
                    vals = plsc.load_gather(rows_o, [rowv73[k] + dd])
                    rowst_v[b, dt_i, di_i, pl.ds(k * 16, 16)] = vals

        idx_copy(0, 0).wait()
        fire_gather(0)
        idx_copy(1, 1)

        def pipe(i2, carry):
            for b in range(2):
                other = 1 - b
                i = i2 * 2 + b
                wait_gather(b)

                @pl.when(i + 1 < gpw)
                def _():
                    wait_idx(i + 1, other)
                    fire_gather(other)

                    @pl.when(i + 2 < gpw)
                    def _():
                        idx_copy(i + 2, b)

                @pl.when(i >= 2)
                def _():
                    wait_out_store(i - 2, b)
                transpose(b)
                out_store(i, b)
            return carry

        lax.fori_loop(0, gpw // 2, pipe, 0)
        wait_out_store(gpw - 2, 0)
        wait_out_store(gpw - 1, 1)

    return gather


def kernel(x, table):
    nb, nh = x.shape
    v, d = table.shape
    bt = nb * nh
    bw = bt // NW
    xt = jnp.transpose(x).astype(jnp.int32).reshape(nh * nb // 128, 128)
    xt_flat = xt.reshape(bt)
    tab_pad = jnp.pad(table, ((0, VOCAB_PAD - v), (0, 0)))
    tab_p = jnp.pad(tab_pad, ((0, 0), (0, ROW_PITCH - d))) if ROW_PITCH > d else tab_pad
    counts = _hist_kernel(bw)(xt_flat)
    ntab = _norm_kernel(d, float(bt) * d)(counts, tab_pad.T, tab_p)
    out5 = _gather_t_kernel(nb, nh, d)(ntab, xt)
    return jnp.transpose(out5, (2, 4, 0, 1, 3)).reshape(nb, nh, d)


# speedup vs baseline: 2.6193x; 1.0250x over previous
import functools

import jax
import jax.numpy as jnp
from jax import lax
from jax.experimental import pallas as pl
from jax.experimental.pallas import tpu as pltpu
from jax.experimental.pallas import tpu_sc as plsc

VOCAB_PAD = 1024
ROW_PITCH = 64
NC = 2
NS = 16
NW = NC * NS
LANES = 16


def _mesh():
    return plsc.VectorSubcoreMesh(
        core_axis_name="c", subcore_axis_name="s",
        num_cores=NC, num_subcores=NS)


def _worker_id():
    return lax.axis_index("s") * NC + lax.axis_index("c")


@functools.lru_cache(maxsize=None)
def _hist_kernel(bw: int):
    @functools.partial(
        pl.kernel,
        out_type=jax.ShapeDtypeStruct((NW, VOCAB_PAD), jnp.float32),
        mesh=_mesh(),
        scratch_types=[
            pltpu.VMEM((bw,), jnp.int32),
            pltpu.VMEM((VOCAB_PAD,), jnp.float32),
        ],
        compiler_params=pltpu.CompilerParams(needs_layout_passes=False),
    )
    def hist(idx_hbm, out_hbm, idx_v, cnt_v):
        wid = _worker_id()
        pltpu.sync_copy(idx_hbm.at[pl.ds(wid * bw, bw)], idx_v)

        def zero_body(i, carry):
            cnt_v[pl.ds(i * LANES, LANES)] = jnp.zeros((LANES,), jnp.float32)
            return carry
        lax.fori_loop(0, VOCAB_PAD // LANES, zero_body, 0)

        ones = jnp.ones((LANES,), jnp.float32)

        @plsc.parallel_loop(0, bw // LANES, step=1, unroll=8)
        def _(i):
            iv = idx_v[pl.ds(i * LANES, LANES)]
            plsc.addupdate_scatter(cnt_v, [iv], ones)

        pltpu.sync_copy(cnt_v, out_hbm.at[wid])

    return hist


@functools.lru_cache(maxsize=None)
def _norm_kernel(d: int, n_elems: float):
    def body(cnt_ref, tab_t_ref, tab_p_ref, out_ref):
        cnt = jnp.sum(cnt_ref[...], axis=0, keepdims=True)
        tab_t = tab_t_ref[...]
        row_sum = jnp.sum(tab_t, axis=0, keepdims=True)
        row_sumsq = jnp.sum(tab_t * tab_t, axis=0, keepdims=True)
        s = jnp.sum(cnt * row_sum)
        q = jnp.sum(cnt * row_sumsq)
        mean = s / n_elems
        var = (q - s * mean) / (n_elems - 1.0)
        scale = lax.rsqrt(var) * (1.0 / 6.0)
        out_ref[...] = jnp.clip(
            (tab_p_ref[...] - mean) * scale + 0.5, 0.0, 1.0)

    return pl.pallas_call(
        body,
        out_shape=jax.ShapeDtypeStruct((VOCAB_PAD, ROW_PITCH), jnp.float32),
    )


@functools.lru_cache(maxsize=None)
def _gather_t_kernel(nb: int, nh: int, d: int):
    ngr = nh * nb // 128
    gpw = ngr // NW
    nbb = nb // 128
    dt = d // 8

    @functools.partial(
        pl.kernel,
        out_type=jax.ShapeDtypeStruct((nh, dt, nbb, 8, 128), jnp.float32),
        mesh=_mesh(),
        scratch_types=[
            pltpu.VMEM((2, 1, 128), jnp.int32),
            pltpu.VMEM((2, 128, ROW_PITCH), jnp.float32),
            pltpu.VMEM((128 * 73,), jnp.float32),
            pltpu.VMEM((2, dt, 8, 128), jnp.float32),
            pltpu.VMEM_SHARED((VOCAB_PAD, ROW_PITCH), jnp.float32),
            pltpu.SemaphoreType.DMA,
            pltpu.SemaphoreType.DMA,
            pltpu.SemaphoreType.DMA,
        ],
        compiler_params=pltpu.CompilerParams(
            needs_layout_passes=False, use_tc_tiling_on_sc=False),
    )
    def gather(ntab_hbm, idx_hbm, out_hbm, idx_v, rows_v, rows_o, rowst_v,
               ntab_sh, isem, gsem, osem):
        wid = _worker_id()
        g0 = wid * gpw

        @pl.when(lax.axis_index("s") == 0)
        def _():
            pltpu.sync_copy(ntab_hbm, ntab_sh)
        plsc.subcore_barrier()
        iota = lax.iota(jnp.int32, 16)
        rowv73 = [(iota + k * 16) * 73 for k in range(8)]

        def idx_copy(i, b):
            return pltpu.async_copy(
                idx_hbm.at[pl.ds(g0 + i, 1)], idx_v.at[b], isem)

        def wait_idx(i, b):
            pltpu.make_async_copy(
                idx_hbm.at[pl.ds(g0 + i, 1)], idx_v.at[b], isem).wait()

        def fire_gather(b):
            pltpu.async_copy(
                ntab_sh.at[idx_v.at[b].at[0]], rows_v.at[b], gsem)

        def wait_gather(b):
            pltpu.make_async_copy(
                ntab_sh.at[idx_v.at[b].at[0]], rows_v.at[b], gsem).wait()

        def grp_hb(g):
            h = (g // (nbb * 8)) * 8 + lax.rem(g, 8)
            bb = lax.rem(g // 8, nbb)
            return h, bb

        def out_store(i, b):
            h, bb = grp_hb(g0 + i)
            return pltpu.async_copy(
                rowst_v.at[b], out_hbm.at[h, :, bb], osem)

        def wait_out_store(i, b):
            h, bb = grp_hb(g0 + i)
            pltpu.make_async_copy(
                rowst_v.at[b], out_hbm.at[h, :, bb], osem).wait()

        def transpose(b):
            src = rows_v.at[b]

            @plsc.parallel_loop(0, 128, step=1, unroll=8)
            def _(r):
                for k in range(4):
                    rows_o[pl.ds(r * 73 + k * 16, 16)] = (
                        src[r, pl.ds(k * 16, 16)])

            @plsc.parallel_loop(0, d, step=1, unroll=4)
            def _(dd):
                dt_i = dd // 8
                di_i = lax.rem(dd, 8)
                for k in range(8):
                    vals = plsc.load_gather(rows_o, [rowv73[k] + dd])
                    rowst_v[b, dt_i, di_i, pl.ds(k * 16, 16)] = vals

        idx_copy(0, 0).wait()
        fire_gather(0)
        idx_copy(1, 1)

        def pipe(i2, carry):
            for b in range(2):
                other = 1 - b
                i = i2 * 2 + b
                wait_gather(b)

                @pl.when(i + 1 < gpw)
                def _():
                    wait_idx(i + 1, other)
                    fire_gather(other)

                    @pl.when(i + 2 < gpw)
                    def _():
                        idx_copy(i + 2, b)

                @pl.when(i >= 2)
                def _():
                    wait_out_store(i - 2, b)
                transpose(b)
                out_store(i, b)
            return carry

        lax.fori_loop(0, gpw // 2, pipe, 0)
        wait_out_store(gpw - 2, 0)
        wait_out_store(gpw - 1, 1)

    return gather


def kernel(x, table):
    nb, nh = x.shape
    v, d = table.shape
    bt = nb * nh
    bw = bt // NW
    xt = (jnp.transpose(x).astype(jnp.int32)
          .reshape(nh // 8, 8, nb // 128, 128)
          .transpose(0, 2, 1, 3)
          .reshape(nh * nb // 128, 128))
    xt_flat = xt.reshape(bt)
    tab_pad = jnp.pad(table, ((0, VOCAB_PAD - v), (0, 0)))
    tab_p = jnp.pad(tab_pad, ((0, 0), (0, ROW_PITCH - d))) if ROW_PITCH > d else tab_pad
    counts = _hist_kernel(bw)(xt_flat)
    ntab = _norm_kernel(d, float(bt) * d)(counts, tab_pad.T, tab_p)
    out5 = _gather_t_kernel(nb, nh, d)(ntab, xt)
    return jnp.transpose(out5, (2, 4, 0, 1, 3)).reshape(nb, nh, d)
